# baseline (speedup 1.0000x reference)
import functools

import jax
import jax.numpy as jnp
from jax import lax
from jax.experimental import pallas as pl
from jax.experimental.pallas import tpu as pltpu
from jax.experimental.pallas import tpu_sc as plsc

_info = plsc.get_sparse_core_info()
_NC = _info.num_cores
_NS = _info.num_subcores
_NW = _NC * _NS


@jax.jit
def _sc_embed(table, idx_cm):
    D = table.shape[1]
    assert D == 64 and idx_cm.shape[0] == 409600
    n_units = 1600
    upw = n_units // _NW

    mesh = plsc.VectorSubcoreMesh(core_axis_name="c", subcore_axis_name="s")

    @functools.partial(
        pl.kernel,
        out_type=jax.ShapeDtypeStruct((1024, 400, 64), jnp.float32),
        mesh=mesh,
        scratch_types=(
            [pltpu.VMEM((256,), jnp.int32) for _ in range(2)]
            + [pltpu.VMEM((256, 64), jnp.float32) for _ in range(2)]
            + [pltpu.SemaphoreType.DMA for _ in range(4)]
        ),
        compiler_params=pltpu.CompilerParams(needs_layout_passes=False),
    )
    def body(table_hbm, idx_hbm, out3, *bufs):
        idxv = bufs[0:2]
        rows = bufs[2:4]
        gsem = bufs[4:6]
        wsem = bufs[6:8]
        wid = lax.axis_index("s") * _NC + lax.axis_index("c")
        base_u = wid * upw

        def stage(k, p):
            u = base_u + k
            cell = u // 4
            bb = u % 4
            pltpu.sync_copy(idx_hbm.at[pl.ds(cell * 1024 + bb * 256, 256)],
                            idxv[p])
            @pl.when(k >= 2)
            def _():
                pltpu.make_async_copy(
                    rows[p], out3.at[pl.ds(0, 256), 0], wsem[p]).wait()

            def grp(g, carry):
                vec = idxv[p][pl.ds(g * 16, 16)]
                for j in range(16):
                    v = vec[j]
                    pltpu.async_copy(
                        table_hbm.at[pl.ds(v, 1), :],
                        rows[p].at[pl.ds(g * 16 + j, 1), :], gsem[p])
                return carry
            lax.fori_loop(0, 16, grp, 0)

        def finish(k, p):
            u = base_u + k
            cell = u // 4
            bb = u % 4
            pltpu.make_async_copy(
                table_hbm.at[pl.ds(0, 256), :], rows[p], gsem[p]).wait()
            pltpu.async_copy(rows[p], out3.at[pl.ds(bb * 256, 256), cell],
                             wsem[p])

        stage(0, 0)

        def pair(k2, carry):
            k = k2 * 2
            stage(k + 1, 1)
            finish(k, 0)

            @pl.when(k2 < upw // 2 - 1)
            def _():
                stage(k + 2, 0)
            finish(k + 1, 1)
            return carry

        lax.fori_loop(0, upw // 2, pair, 0)
        for p in range(2):
            pltpu.make_async_copy(
                rows[p], out3.at[pl.ds(0, 256), 0], wsem[p]).wait()

    return body(table, idx_cm)


def kernel(x, table):
    assert x.ndim == 4, f"TokenEmbedding expects 4D [B,H,W,C], got {x.shape}"
    vocab, dim = table.shape
    if x.shape[-1] == vocab:
        idx = jnp.argmax(x, axis=-1).astype(jnp.int32)
    else:
        idx = x.astype(jnp.int32)
    B, H, W = x.shape[0], x.shape[1], x.shape[2]
    idx_cm = idx.reshape(B, H * W).T.reshape(-1)
    out3 = _sc_embed(table, idx_cm)
    out = out3.reshape(B, H, W, dim)
    return out
